# baseline (speedup 1.0000x reference)
import functools

import jax
import jax.numpy as jnp
from jax import lax
from jax.experimental import pallas as pl
from jax.experimental.pallas import tpu as pltpu

N = 4096
D = 512
K = 32
RB = 512
NEG_INF = float("-inf")


def _normalize_body(x_ref, out_ref):
    x = x_ref[...]
    sq = x * x
    p = sq[:, 0:128] + sq[:, 128:256]
    p = p + sq[:, 256:384]
    p = p + sq[:, 384:512]
    acc = p
    for j in range(1, 16):
        acc = acc + pltpu.roll(p, 128 - 8 * j, 1)
    t = acc + pltpu.roll(acc, 124, 1)
    t = t + pltpu.roll(t, 126, 1)
    t = t + pltpu.roll(t, 127, 1)
    n2 = t[:, 0:1]
    s = n2 * lax.rsqrt(n2)
    s = jnp.where(n2 == 0.0, 0.0, s)
    norm = jnp.maximum(s, 1e-12)
    out_ref[...] = x * (1.0 / norm)


def _topk_body(x_rows_ref, x_all_ref, adj_ref, vals_ref, idx_ref, cand_ref):
    x = x_rows_ref[...]
    x_all = x_all_ref[...]
    adj = adj_ref[...]
    sim = lax.dot_general(
        x.astype(jnp.bfloat16), x_all.astype(jnp.bfloat16),
        (((1,), (1,)), ((), ())),
        preferred_element_type=jnp.float32,
    )
    mask = adj > 0.0
    w = jnp.where(mask, sim * adj, 0.0)
    rs = jnp.sum(w, axis=1, keepdims=True)
    probs = w / rs
    cand_ref[...] = jnp.where(mask, probs, NEG_INF)

    col = lax.broadcasted_iota(jnp.int32, (RB, N), 1)
    kcol = lax.broadcasted_iota(jnp.int32, (RB, K), 1)

    def step(t, carry):
        vals, idxs = carry
        c = cand_ref[...]
        m = jnp.max(c, axis=1, keepdims=True)
        sel = jnp.min(jnp.where(c == m, col, N), axis=1, keepdims=True)
        cand_ref[...] = jnp.where(col == sel, NEG_INF, c)
        vals = jnp.where(kcol == t, m, vals)
        idxs = jnp.where(kcol == t, sel, idxs)
        return vals, idxs

    vals0 = jnp.zeros((RB, K), jnp.float32)
    idxs0 = jnp.zeros((RB, K), jnp.int32)
    vals, idxs = lax.fori_loop(0, K, step, (vals0, idxs0))
    vals_ref[...] = vals
    idx_ref[...] = idxs


def kernel(adjacency_matrix, transaction_record, labels):
    del labels
    x_norm = pl.pallas_call(
        _normalize_body,
        grid=(N // 512,),
        in_specs=[pl.BlockSpec((512, D), lambda i: (i, 0))],
        out_specs=pl.BlockSpec((512, D), lambda i: (i, 0)),
        out_shape=jax.ShapeDtypeStruct((N, D), jnp.float32),
    )(transaction_record)

    vals, idxs = pl.pallas_call(
        _topk_body,
        grid=(N // RB,),
        in_specs=[
            pl.BlockSpec((RB, D), lambda i: (i, 0)),
            pl.BlockSpec((N, D), lambda i: (0, 0)),
            pl.BlockSpec((RB, N), lambda i: (i, 0)),
        ],
        out_specs=[
            pl.BlockSpec((RB, K), lambda i: (i, 0)),
            pl.BlockSpec((RB, K), lambda i: (i, 0)),
        ],
        out_shape=[
            jax.ShapeDtypeStruct((N, K), jnp.float32),
            jax.ShapeDtypeStruct((N, K), jnp.int32),
        ],
        scratch_shapes=[pltpu.VMEM((RB, N), jnp.float32)],
    )(x_norm, x_norm, adjacency_matrix)
    return vals, idxs
